# Initial kernel scaffold; baseline (speedup 1.0000x reference)
import jax
import jax.numpy as jnp
from jax.experimental import pallas as pl


def kernel(x, output_tensor_mask):
    raise NotImplementedError("write your pallas kernel here")


# speedup vs baseline: 1.9653x; 1.9653x over previous
import functools

import jax
import jax.numpy as jnp
from jax.experimental import pallas as pl


def _onehot_gather_body(idx_ref, x_ref, o_ref):
    k = idx_ref.shape[-1]
    f = x_ref.shape[-1]
    iota = jax.lax.broadcasted_iota(jnp.int32, (f, k), 0)
    sel = (iota == idx_ref[...]).astype(jnp.bfloat16)
    xv = x_ref[...]
    xh = xv.astype(jnp.bfloat16)
    xl = (xv - xh.astype(jnp.float32)).astype(jnp.bfloat16)
    acc = jax.lax.dot_general(
        xh, sel, (((1,), (0,)), ((), ())), preferred_element_type=jnp.float32
    )
    acc += jax.lax.dot_general(
        xl, sel, (((1,), (0,)), ((), ())), preferred_element_type=jnp.float32
    )
    o_ref[...] = acc


@jax.jit
def kernel(x, output_tensor_mask):
    b, s, f = x.shape
    k = output_tensor_mask.shape[0]
    rows = b * s
    tile = 256
    x2 = x.reshape(rows, f)
    idx = output_tensor_mask.reshape(1, k)
    out = pl.pallas_call(
        _onehot_gather_body,
        grid=(rows // tile,),
        in_specs=[
            pl.BlockSpec((1, k), lambda i: (0, 0)),
            pl.BlockSpec((tile, f), lambda i: (i, 0)),
        ],
        out_specs=pl.BlockSpec((tile, k), lambda i: (i, 0)),
        out_shape=jax.ShapeDtypeStruct((rows, k), jnp.float32),
    )(idx, x2)
    return out.reshape(b, s, k)
